# baseline (speedup 1.0000x reference)
import functools

import jax
import jax.numpy as jnp
from jax.experimental import pallas as pl
from jax.experimental.pallas import tpu as pltpu

_NEG = -1e30


def _ln(x, g, b, eps=1e-5):
    m = jnp.mean(x, axis=-1, keepdims=True)
    d = x - m
    v = jnp.mean(d * d, axis=-1, keepdims=True)
    return d * jax.lax.rsqrt(v + eps) * g + b


def _dot(a, b):
    return jnp.dot(a, b, preferred_element_type=jnp.float32)


def _gnn_kernel(
    x_ref, adj_ref,
    ne_w1t_ref, ne_b1_ref, ne_g1_ref, ne_be1_ref,
    ne_w2t_ref, ne_b2_ref, ne_g2_ref, ne_be2_ref,
    wnode_ref, bnode_ref, mm_g1_ref, mm_be1_ref,
    mm_w2t_ref, mm_b2_ref,
    wct_ref, at_w2_ref, at_b2_ref,
    gru_wit_ref, gru_bi_ref,
    op_wt_ref, op_b_ref,
    out_ref,
):
    T = adj_ref.shape[0]
    H = mm_g1_ref.shape[-1]

    x = x_ref[0]
    adj = adj_ref[...]
    mask = adj > 0.0
    maskf = mask.astype(jnp.float32)

    h = _dot(x, ne_w1t_ref[...]) + ne_b1_ref[...]
    h = jnp.maximum(_ln(h, ne_g1_ref[...], ne_be1_ref[...]), 0.0)
    h = _dot(h, ne_w2t_ref[...]) + ne_b2_ref[...]
    h = jnp.maximum(_ln(h, ne_g2_ref[...], ne_be2_ref[...]), 0.0)

    mm_g1 = mm_g1_ref[...]
    mm_be1 = mm_be1_ref[...]
    at_w2 = at_w2_ref[...]
    at_b2 = at_b2_ref[0, 0]
    inv_h = 1.0 / H

    for _ in range(3):
        big = _dot(h, wnode_ref[...]) + bnode_ref[...]
        a = big[:, :H]
        b = big[:, H:2 * H]
        c = big[:, 2 * H:3 * H]
        gh = big[:, 3 * H:]

        da = a - jnp.mean(a, axis=-1, keepdims=True)
        db = b - jnp.mean(b, axis=-1, keepdims=True)
        na = jnp.sum(da * da, axis=-1, keepdims=True) * inv_h
        nb = jnp.sum(db * db, axis=-1, keepdims=True) * inv_h
        gram = jax.lax.dot_general(
            da, db, (((1,), (1,)), ((), ())),
            preferred_element_type=jnp.float32) * (2.0 * inv_h)
        v = na + gram + nb.T
        r = jax.lax.rsqrt(v + 1e-5)

        dag = da * mm_g1
        dbg = db * mm_g1
        u = jnp.maximum(
            (dag[:, None, :] + dbg[None, :, :]) * r[:, :, None] + mm_be1,
            0.0)

        u2 = u.reshape(T * T, H)
        t = _dot(u2, wct_ref[...]).reshape(T, T, H) + c[None, :, :]
        w = jnp.sum(jnp.tanh(t) * at_w2[None, :, :], axis=-1) + at_b2

        wl = jnp.where(mask, w, _NEG)
        p = jnp.exp(wl - jnp.max(wl, axis=0, keepdims=True))
        attn = p / jnp.sum(p, axis=0, keepdims=True) * maskf

        s = jnp.sum(attn[:, :, None] * u, axis=0)
        colsum = jnp.sum(attn.T, axis=-1, keepdims=True)
        agg = _dot(s, mm_w2t_ref[...]) + colsum * mm_b2_ref[...]

        gi = _dot(agg, gru_wit_ref[...]) + gru_bi_ref[...]
        rg = jax.nn.sigmoid(gi[:, :H] + gh[:, :H])
        z = jax.nn.sigmoid(gi[:, H:2 * H] + gh[:, H:2 * H])
        n = jnp.tanh(gi[:, 2 * H:] + rg * gh[:, 2 * H:])
        h = (1.0 - z) * n + z * h

    out_ref[0] = _dot(h, op_wt_ref[...]) + op_b_ref[...]


@jax.jit
def kernel(node_embeddings, adjacency_matrix,
           ne_w1, ne_b1, ne_g1, ne_be1, ne_w2, ne_b2, ne_g2, ne_be2,
           mm_w1, mm_b1, mm_g1, mm_be1, mm_w2, mm_b2,
           at_w1, at_b1, at_w2, at_b2,
           gru_wi, gru_bi, gru_wh, gru_bh,
           op_w, op_b):
    B, T, E = node_embeddings.shape
    H = ne_b1.shape[0]

    w1a = mm_w1[:, :H]
    w1b = mm_w1[:, H:]
    at_w1a = at_w1[:, :H]
    at_w1b = at_w1[:, H:]
    wc = at_w1b @ mm_w2
    att_bias = at_b1 + at_w1b @ mm_b2

    wnode = jnp.concatenate([w1a.T, w1b.T, at_w1a.T, gru_wh.T], axis=1)
    bnode = jnp.concatenate(
        [jnp.zeros_like(mm_b1), mm_b1, att_bias, gru_bh])[None, :]

    row = lambda v: v[None, :]
    args = (
        node_embeddings, adjacency_matrix,
        ne_w1.T, row(ne_b1), row(ne_g1), row(ne_be1),
        ne_w2.T, row(ne_b2), row(ne_g2), row(ne_be2),
        wnode, bnode, row(mm_g1), row(mm_be1),
        mm_w2.T, row(mm_b2),
        wc.T, at_w2, at_b2[None, :],
        gru_wi.T, row(gru_bi),
        op_w.T, row(op_b),
    )

    fixed = lambda shape: pl.BlockSpec(shape, lambda b: (0,) * len(shape))
    in_specs = [
        pl.BlockSpec((1, T, E), lambda b: (b, 0, 0)),
        fixed((T, T)),
    ] + [fixed(a.shape) for a in args[2:]]

    return pl.pallas_call(
        _gnn_kernel,
        grid=(B,),
        in_specs=in_specs,
        out_specs=pl.BlockSpec((1, T, E), lambda b: (b, 0, 0)),
        out_shape=jax.ShapeDtypeStruct((B, T, E), jnp.float32),
        compiler_params=pltpu.CompilerParams(
            dimension_semantics=("arbitrary",),
        ),
    )(*args)


# speedup vs baseline: 1.6071x; 1.6071x over previous
import functools

import jax
import jax.numpy as jnp
from jax.experimental import pallas as pl
from jax.experimental.pallas import tpu as pltpu

_NEG = -1e30


def _ln(x, g, b, eps=1e-5):
    m = jnp.mean(x, axis=-1, keepdims=True)
    d = x - m
    v = jnp.mean(d * d, axis=-1, keepdims=True)
    return d * jax.lax.rsqrt(v + eps) * g + b


def _dot(a, b):
    return jnp.dot(a, b, preferred_element_type=jnp.float32)


def _gnn_kernel(
    x_ref, adj_ref,
    ne_w1t_ref, ne_b1_ref, ne_g1_ref, ne_be1_ref,
    ne_w2t_ref, ne_b2_ref, ne_g2_ref, ne_be2_ref,
    wnode_ref, bnode_ref, mm_g1_ref, mm_be1_ref,
    mm_w2t_ref, mm_b2_ref,
    wct_ref, at_w2_ref, at_b2_ref,
    gru_wit_ref, gru_bi_ref,
    op_wt_ref, op_b_ref,
    out_ref,
):
    T = adj_ref.shape[0]
    H = mm_g1_ref.shape[-1]
    BB = x_ref.shape[0]
    E = x_ref.shape[-1]

    x = x_ref[...].reshape(BB * T, E)
    adj = adj_ref[...]
    mask = (adj > 0.0)[None, :, :]
    maskf = mask.astype(jnp.float32)

    h = _dot(x, ne_w1t_ref[...]) + ne_b1_ref[...]
    h = jnp.maximum(_ln(h, ne_g1_ref[...], ne_be1_ref[...]), 0.0)
    h = _dot(h, ne_w2t_ref[...]) + ne_b2_ref[...]
    h = jnp.maximum(_ln(h, ne_g2_ref[...], ne_be2_ref[...]), 0.0)

    mm_g1 = mm_g1_ref[...]
    mm_be1 = mm_be1_ref[...]
    at_w2 = at_w2_ref[...]
    at_b2 = at_b2_ref[0, 0]
    inv_h = 1.0 / H

    for _ in range(3):
        big = _dot(h, wnode_ref[...]) + bnode_ref[...]
        a = big[:, :H]
        b = big[:, H:2 * H]
        c = big[:, 2 * H:3 * H]
        gh = big[:, 3 * H:]

        da = a - jnp.mean(a, axis=-1, keepdims=True)
        db = b - jnp.mean(b, axis=-1, keepdims=True)
        na = jnp.sum(da * da, axis=-1, keepdims=True) * inv_h
        nb = jnp.sum(db * db, axis=-1, keepdims=True) * inv_h
        da3 = da.reshape(BB, T, H)
        db3 = db.reshape(BB, T, H)
        gram = jax.lax.dot_general(
            da3, db3, (((2,), (2,)), ((0,), (0,))),
            preferred_element_type=jnp.float32) * (2.0 * inv_h)
        v = (na.reshape(BB, T, 1) + gram + nb.reshape(BB, 1, T))
        r = jax.lax.rsqrt(v + 1e-5)

        dag = (da * mm_g1).reshape(BB, T, 1, H)
        dbg = (db * mm_g1).reshape(BB, 1, T, H)
        u = jnp.maximum(
            (dag + dbg) * r[:, :, :, None] + mm_be1,
            0.0)

        u2 = u.reshape(BB * T * T, H)
        t = (_dot(u2, wct_ref[...]).reshape(BB, T, T, H)
             + c.reshape(BB, 1, T, H))
        w = jnp.sum(jnp.tanh(t) * at_w2[None, None, :, :], axis=-1) + at_b2

        wl = jnp.where(mask, w, _NEG)
        p = jnp.exp(wl - jnp.max(wl, axis=1, keepdims=True))
        attn = p / jnp.sum(p, axis=1, keepdims=True) * maskf

        s = jnp.sum(attn[:, :, :, None] * u, axis=1)
        colsum = jnp.sum(jnp.swapaxes(attn, 1, 2), axis=-1, keepdims=True)
        agg = (_dot(s.reshape(BB * T, H), mm_w2t_ref[...])
               + colsum.reshape(BB * T, 1) * mm_b2_ref[...])

        gi = _dot(agg, gru_wit_ref[...]) + gru_bi_ref[...]
        rg = jax.nn.sigmoid(gi[:, :H] + gh[:, :H])
        z = jax.nn.sigmoid(gi[:, H:2 * H] + gh[:, H:2 * H])
        n = jnp.tanh(gi[:, 2 * H:] + rg * gh[:, 2 * H:])
        h = (1.0 - z) * n + z * h

    out = _dot(h, op_wt_ref[...]) + op_b_ref[...]
    out_ref[...] = out.reshape(BB, T, E)


@jax.jit
def kernel(node_embeddings, adjacency_matrix,
           ne_w1, ne_b1, ne_g1, ne_be1, ne_w2, ne_b2, ne_g2, ne_be2,
           mm_w1, mm_b1, mm_g1, mm_be1, mm_w2, mm_b2,
           at_w1, at_b1, at_w2, at_b2,
           gru_wi, gru_bi, gru_wh, gru_bh,
           op_w, op_b):
    B, T, E = node_embeddings.shape
    H = ne_b1.shape[0]

    w1a = mm_w1[:, :H]
    w1b = mm_w1[:, H:]
    at_w1a = at_w1[:, :H]
    at_w1b = at_w1[:, H:]
    wc = at_w1b @ mm_w2
    att_bias = at_b1 + at_w1b @ mm_b2

    wnode = jnp.concatenate([w1a.T, w1b.T, at_w1a.T, gru_wh.T], axis=1)
    bnode = jnp.concatenate(
        [jnp.zeros_like(mm_b1), mm_b1, att_bias, gru_bh])[None, :]

    row = lambda v: v[None, :]
    args = (
        node_embeddings, adjacency_matrix,
        ne_w1.T, row(ne_b1), row(ne_g1), row(ne_be1),
        ne_w2.T, row(ne_b2), row(ne_g2), row(ne_be2),
        wnode, bnode, row(mm_g1), row(mm_be1),
        mm_w2.T, row(mm_b2),
        wc.T, at_w2, at_b2[None, :],
        gru_wi.T, row(gru_bi),
        op_w.T, row(op_b),
    )

    BB = 4
    fixed = lambda shape: pl.BlockSpec(shape, lambda b: (0,) * len(shape))
    in_specs = [
        pl.BlockSpec((BB, T, E), lambda b: (b, 0, 0)),
        fixed((T, T)),
    ] + [fixed(a.shape) for a in args[2:]]

    return pl.pallas_call(
        _gnn_kernel,
        grid=(B // BB,),
        in_specs=in_specs,
        out_specs=pl.BlockSpec((BB, T, E), lambda b: (b, 0, 0)),
        out_shape=jax.ShapeDtypeStruct((B, T, E), jnp.float32),
        compiler_params=pltpu.CompilerParams(
            dimension_semantics=("arbitrary",),
        ),
    )(*args)


# speedup vs baseline: 2.0795x; 1.2940x over previous
import functools

import jax
import jax.numpy as jnp
from jax.experimental import pallas as pl
from jax.experimental.pallas import tpu as pltpu

_NEG = -1e30


def _ln(x, g, b, eps=1e-5):
    m = jnp.mean(x, axis=-1, keepdims=True)
    d = x - m
    v = jnp.mean(d * d, axis=-1, keepdims=True)
    return d * jax.lax.rsqrt(v + eps) * g + b


def _dot(a, b):
    return jnp.dot(a, b, preferred_element_type=jnp.float32)


def _gnn_kernel(
    x_ref, adj_ref,
    ne_w1t_ref, ne_b1_ref, ne_g1_ref, ne_be1_ref,
    ne_w2t_ref, ne_b2_ref, ne_g2_ref, ne_be2_ref,
    wnode_ref, bnode_ref, mm_g1_ref, mm_be1_ref,
    mm_w2t_ref, mm_b2_ref,
    wct_ref, at_w2_ref, at_b2_ref,
    gru_wit_ref, gru_bi_ref,
    op_wt_ref, op_b_ref,
    out_ref,
):
    T = adj_ref.shape[0]
    H = mm_g1_ref.shape[-1]
    BB = x_ref.shape[0]
    E = x_ref.shape[-1]

    x = x_ref[...].reshape(BB * T, E)
    adj = adj_ref[...]
    mask = (adj > 0.0)[None, :, :]
    maskf = mask.astype(jnp.float32)

    h = _dot(x, ne_w1t_ref[...]) + ne_b1_ref[...]
    h = jnp.maximum(_ln(h, ne_g1_ref[...], ne_be1_ref[...]), 0.0)
    h = _dot(h, ne_w2t_ref[...]) + ne_b2_ref[...]
    h = jnp.maximum(_ln(h, ne_g2_ref[...], ne_be2_ref[...]), 0.0)

    mm_g1 = mm_g1_ref[...]
    mm_be1 = mm_be1_ref[...]
    at_w2 = at_w2_ref[...]
    at_b2 = at_b2_ref[0, 0]
    inv_h = 1.0 / H

    for _ in range(3):
        big = _dot(h, wnode_ref[...]) + bnode_ref[...]
        a = big[:, :H]
        b = big[:, H:2 * H]
        c = big[:, 2 * H:3 * H]
        gh = big[:, 3 * H:]

        da = a - jnp.mean(a, axis=-1, keepdims=True)
        db = b - jnp.mean(b, axis=-1, keepdims=True)
        na = jnp.sum(da * da, axis=-1, keepdims=True) * inv_h
        nb = jnp.sum(db * db, axis=-1, keepdims=True) * inv_h
        da3 = da.reshape(BB, T, H)
        db3 = db.reshape(BB, T, H)
        gram = jax.lax.dot_general(
            da3, db3, (((2,), (2,)), ((0,), (0,))),
            preferred_element_type=jnp.float32) * (2.0 * inv_h)
        v = (na.reshape(BB, T, 1) + gram + nb.reshape(BB, 1, T))
        r = jax.lax.rsqrt(v + 1e-5)

        dag = (da * mm_g1).reshape(BB, T, 1, H)
        dbg = (db * mm_g1).reshape(BB, 1, T, H)
        u = jnp.maximum(
            (dag + dbg) * r[:, :, :, None] + mm_be1,
            0.0)

        u2 = u.reshape(BB * T * T, H)
        t = (_dot(u2, wct_ref[...]).reshape(BB, T, T, H)
             + c.reshape(BB, 1, T, H))
        w = jnp.sum(jnp.tanh(t) * at_w2[None, None, :, :], axis=-1) + at_b2

        wl = jnp.where(mask, w, _NEG)
        p = jnp.exp(wl - jnp.max(wl, axis=1, keepdims=True))
        attn = p / jnp.sum(p, axis=1, keepdims=True) * maskf

        s = jnp.sum(attn[:, :, :, None] * u, axis=1)
        colsum = jnp.sum(jnp.swapaxes(attn, 1, 2), axis=-1, keepdims=True)
        agg = (_dot(s.reshape(BB * T, H), mm_w2t_ref[...])
               + colsum.reshape(BB * T, 1) * mm_b2_ref[...])

        gi = _dot(agg, gru_wit_ref[...]) + gru_bi_ref[...]
        rg = jax.nn.sigmoid(gi[:, :H] + gh[:, :H])
        z = jax.nn.sigmoid(gi[:, H:2 * H] + gh[:, H:2 * H])
        n = jnp.tanh(gi[:, 2 * H:] + rg * gh[:, 2 * H:])
        h = (1.0 - z) * n + z * h

    out = _dot(h, op_wt_ref[...]) + op_b_ref[...]
    out_ref[...] = out.reshape(BB, T, E)


@jax.jit
def kernel(node_embeddings, adjacency_matrix,
           ne_w1, ne_b1, ne_g1, ne_be1, ne_w2, ne_b2, ne_g2, ne_be2,
           mm_w1, mm_b1, mm_g1, mm_be1, mm_w2, mm_b2,
           at_w1, at_b1, at_w2, at_b2,
           gru_wi, gru_bi, gru_wh, gru_bh,
           op_w, op_b):
    B, T, E = node_embeddings.shape
    H = ne_b1.shape[0]

    w1a = mm_w1[:, :H]
    w1b = mm_w1[:, H:]
    at_w1a = at_w1[:, :H]
    at_w1b = at_w1[:, H:]
    wc = at_w1b @ mm_w2
    att_bias = at_b1 + at_w1b @ mm_b2

    wnode = jnp.concatenate([w1a.T, w1b.T, at_w1a.T, gru_wh.T], axis=1)
    bnode = jnp.concatenate(
        [jnp.zeros_like(mm_b1), mm_b1, att_bias, gru_bh])[None, :]

    row = lambda v: v[None, :]
    args = (
        node_embeddings, adjacency_matrix,
        ne_w1.T, row(ne_b1), row(ne_g1), row(ne_be1),
        ne_w2.T, row(ne_b2), row(ne_g2), row(ne_be2),
        wnode, bnode, row(mm_g1), row(mm_be1),
        mm_w2.T, row(mm_b2),
        wc.T, at_w2, at_b2[None, :],
        gru_wi.T, row(gru_bi),
        op_w.T, row(op_b),
    )

    BB = 8
    fixed = lambda shape: pl.BlockSpec(shape, lambda b: (0,) * len(shape))
    in_specs = [
        pl.BlockSpec((BB, T, E), lambda b: (b, 0, 0)),
        fixed((T, T)),
    ] + [fixed(a.shape) for a in args[2:]]

    return pl.pallas_call(
        _gnn_kernel,
        grid=(B // BB,),
        in_specs=in_specs,
        out_specs=pl.BlockSpec((BB, T, E), lambda b: (b, 0, 0)),
        out_shape=jax.ShapeDtypeStruct((B, T, E), jnp.float32),
        compiler_params=pltpu.CompilerParams(
            dimension_semantics=("arbitrary",),
        ),
    )(*args)


# speedup vs baseline: 2.1729x; 1.0449x over previous
import functools

import jax
import jax.numpy as jnp
from jax.experimental import pallas as pl
from jax.experimental.pallas import tpu as pltpu

_NEG = -1e30


def _ln(x, g, b, eps=1e-5):
    m = jnp.mean(x, axis=-1, keepdims=True)
    d = x - m
    v = jnp.mean(d * d, axis=-1, keepdims=True)
    return d * jax.lax.rsqrt(v + eps) * g + b


def _dot(a, b):
    return jnp.dot(a, b, preferred_element_type=jnp.float32)


def _gnn_kernel(
    x_ref, adj_ref,
    ne_w1t_ref, ne_b1_ref, ne_g1_ref, ne_be1_ref,
    ne_w2t_ref, ne_b2_ref, ne_g2_ref, ne_be2_ref,
    wnode_ref, bnode_ref, mm_g1_ref, mm_be1_ref,
    mm_w2t_ref, mm_b2_ref,
    wct_ref, at_w2_ref, at_b2_ref,
    gru_wit_ref, gru_bi_ref,
    op_wt_ref, op_b_ref,
    out_ref,
):
    T = adj_ref.shape[0]
    H = mm_g1_ref.shape[-1]
    BB = x_ref.shape[0]
    E = x_ref.shape[-1]

    x = x_ref[...].reshape(BB * T, E)
    adj = adj_ref[...]
    mask = (adj > 0.0)[None, :, :]
    maskf = mask.astype(jnp.float32)

    h = _dot(x, ne_w1t_ref[...]) + ne_b1_ref[...]
    h = jnp.maximum(_ln(h, ne_g1_ref[...], ne_be1_ref[...]), 0.0)
    h = _dot(h, ne_w2t_ref[...]) + ne_b2_ref[...]
    h = jnp.maximum(_ln(h, ne_g2_ref[...], ne_be2_ref[...]), 0.0)

    mm_g1 = mm_g1_ref[...]
    mm_be1 = mm_be1_ref[...]
    at_w2 = at_w2_ref[...]
    at_b2 = at_b2_ref[0, 0]
    inv_h = 1.0 / H

    for _ in range(3):
        big = _dot(h, wnode_ref[...]) + bnode_ref[...]
        a = big[:, :H]
        b = big[:, H:2 * H]
        c = big[:, 2 * H:3 * H]
        gh = big[:, 3 * H:]

        da = a - jnp.mean(a, axis=-1, keepdims=True)
        db = b - jnp.mean(b, axis=-1, keepdims=True)
        na = jnp.sum(da * da, axis=-1, keepdims=True) * inv_h
        nb = jnp.sum(db * db, axis=-1, keepdims=True) * inv_h
        da3 = da.reshape(BB, T, H)
        db3 = db.reshape(BB, T, H)
        gram = jax.lax.dot_general(
            da3, db3, (((2,), (2,)), ((0,), (0,))),
            preferred_element_type=jnp.float32) * (2.0 * inv_h)
        v = (na.reshape(BB, T, 1) + gram + nb.reshape(BB, 1, T))
        r = jax.lax.rsqrt(v + 1e-5)

        dag = (da * mm_g1).reshape(BB, T, 1, H)
        dbg = (db * mm_g1).reshape(BB, 1, T, H)
        u = jnp.maximum(
            (dag + dbg) * r[:, :, :, None] + mm_be1,
            0.0)

        u2 = u.reshape(BB * T * T, H)
        t = (_dot(u2, wct_ref[...]).reshape(BB, T, T, H)
             + c.reshape(BB, 1, T, H))
        w = jnp.sum(jnp.tanh(t) * at_w2[None, None, :, :], axis=-1) + at_b2

        wl = jnp.where(mask, w, _NEG)
        p = jnp.exp(wl - jnp.max(wl, axis=1, keepdims=True))
        attn = p / jnp.sum(p, axis=1, keepdims=True) * maskf

        s = jnp.sum(attn[:, :, :, None] * u, axis=1)
        colsum = jnp.sum(jnp.swapaxes(attn, 1, 2), axis=-1, keepdims=True)
        agg = (_dot(s.reshape(BB * T, H), mm_w2t_ref[...])
               + colsum.reshape(BB * T, 1) * mm_b2_ref[...])

        gi = _dot(agg, gru_wit_ref[...]) + gru_bi_ref[...]
        rg = jax.nn.sigmoid(gi[:, :H] + gh[:, :H])
        z = jax.nn.sigmoid(gi[:, H:2 * H] + gh[:, H:2 * H])
        n = jnp.tanh(gi[:, 2 * H:] + rg * gh[:, 2 * H:])
        h = (1.0 - z) * n + z * h

    out = _dot(h, op_wt_ref[...]) + op_b_ref[...]
    out_ref[...] = out.reshape(BB, T, E)


@jax.jit
def kernel(node_embeddings, adjacency_matrix,
           ne_w1, ne_b1, ne_g1, ne_be1, ne_w2, ne_b2, ne_g2, ne_be2,
           mm_w1, mm_b1, mm_g1, mm_be1, mm_w2, mm_b2,
           at_w1, at_b1, at_w2, at_b2,
           gru_wi, gru_bi, gru_wh, gru_bh,
           op_w, op_b):
    B, T, E = node_embeddings.shape
    H = ne_b1.shape[0]

    w1a = mm_w1[:, :H]
    w1b = mm_w1[:, H:]
    at_w1a = at_w1[:, :H]
    at_w1b = at_w1[:, H:]
    wc = at_w1b @ mm_w2
    att_bias = at_b1 + at_w1b @ mm_b2

    wnode = jnp.concatenate([w1a.T, w1b.T, at_w1a.T, gru_wh.T], axis=1)
    bnode = jnp.concatenate(
        [jnp.zeros_like(mm_b1), mm_b1, att_bias, gru_bh])[None, :]

    row = lambda v: v[None, :]
    args = (
        node_embeddings, adjacency_matrix,
        ne_w1.T, row(ne_b1), row(ne_g1), row(ne_be1),
        ne_w2.T, row(ne_b2), row(ne_g2), row(ne_be2),
        wnode, bnode, row(mm_g1), row(mm_be1),
        mm_w2.T, row(mm_b2),
        wc.T, at_w2, at_b2[None, :],
        gru_wi.T, row(gru_bi),
        op_w.T, row(op_b),
    )

    BB = 16
    fixed = lambda shape: pl.BlockSpec(shape, lambda b: (0,) * len(shape))
    in_specs = [
        pl.BlockSpec((BB, T, E), lambda b: (b, 0, 0)),
        fixed((T, T)),
    ] + [fixed(a.shape) for a in args[2:]]

    return pl.pallas_call(
        _gnn_kernel,
        grid=(B // BB,),
        in_specs=in_specs,
        out_specs=pl.BlockSpec((BB, T, E), lambda b: (b, 0, 0)),
        out_shape=jax.ShapeDtypeStruct((B, T, E), jnp.float32),
        compiler_params=pltpu.CompilerParams(
            dimension_semantics=("arbitrary",),
        ),
    )(*args)
